# Initial kernel scaffold; baseline (speedup 1.0000x reference)
import functools

import jax
import jax.numpy as jnp
from jax import lax
from jax.experimental import pallas as pl
from jax.experimental.pallas import tpu as pltpu
from jax.experimental.pallas import tpu_sc as plsc

B = 16384
VOCABS = (1000, 8, 30, 40, 8, 4, 20, 20, 4, 10000, 7)
DIMS = (100, 1, 3, 4, 1, 1, 2, 2, 1, 1000, 1)
OFFS = (0, 100, 101, 104, 108, 109, 110, 112, 114, 115, 1115)
NT = 11
NP = 16

NC, NS = 2, 16
NW = NC * NS
BPW = B // NW

T9_BLOCK = 2000
T9_GRID = VOCABS[9] // T9_BLOCK

_F32 = jnp.float32
_HIGH = lax.Precision.HIGHEST


def _proj_body(*refs):
    t_refs = refs[:NT]
    w_refs = refs[NT:2 * NT]
    p_refs = refs[2 * NT:]
    step = pl.program_id(0)

    def project(t):
        tbl = t_refs[t][...]
        d = DIMS[t]
        if d >= 8:
            return lax.dot_general(tbl, w_refs[t][...], (((1,), (0,)), ((), ())),
                                   preferred_element_type=_F32, precision=_HIGH)
        acc = tbl[:, 0:1] * w_refs[t][0:1, :]
        for k in range(1, d):
            acc = acc + tbl[:, k:k + 1] * w_refs[t][k:k + 1, :]
        return acc

    p_refs[9][...] = project(9)

    @pl.when(step == 0)
    def _():
        for t in range(NT):
            if t != 9:
                p_refs[t][...] = project(t)


def _run_projection(tables, w_slices):
    in_specs = []
    for t in range(NT):
        if t == 9:
            in_specs.append(pl.BlockSpec((T9_BLOCK, DIMS[9]), lambda i: (i, 0)))
        else:
            in_specs.append(pl.BlockSpec(tables[t].shape, lambda i: (0, 0)))
    for t in range(NT):
        in_specs.append(pl.BlockSpec(w_slices[t].shape, lambda i: (0, 0)))
    out_specs = []
    out_shapes = []
    for t in range(NT):
        out_shapes.append(jax.ShapeDtypeStruct((VOCABS[t], NP), _F32))
        if t == 9:
            out_specs.append(pl.BlockSpec((T9_BLOCK, NP), lambda i: (i, 0)))
        else:
            out_specs.append(pl.BlockSpec((VOCABS[t], NP), lambda i: (0, 0)))
    return pl.pallas_call(
        _proj_body,
        grid=(T9_GRID,),
        in_specs=in_specs,
        out_specs=out_specs,
        out_shape=out_shapes,
    )(*tables, *w_slices)


_UNROLL = 8


def _gather_body(*refs):
    p_refs = refs[:NT]
    i_refs = refs[NT:2 * NT]
    out_hbm = refs[2 * NT]
    idx_v, acc_v, rows_v, sem = refs[2 * NT + 1:]

    wid = lax.axis_index("s") * NC + lax.axis_index("c")
    base = wid * BPW

    pltpu.sync_copy(i_refs[0].at[pl.ds(base, BPW)], idx_v)
    pltpu.async_copy(p_refs[0].at[idx_v], acc_v, sem).wait()

    for t in range(1, NT):
        pltpu.sync_copy(i_refs[t].at[pl.ds(base, BPW)], idx_v)
        pltpu.async_copy(p_refs[t].at[idx_v], rows_v, sem).wait()

        def body(i, _):
            b0 = i * _UNROLL
            for u in range(_UNROLL):
                plsc.addupdate(acc_v.at[b0 + u], rows_v[b0 + u])
            return 0

        lax.fori_loop(0, BPW // _UNROLL, body, 0)

    pltpu.sync_copy(acc_v, out_hbm.at[pl.ds(base, BPW)])


_gather_sum = functools.partial(
    pl.kernel,
    out_type=jax.ShapeDtypeStruct((B, NP), _F32),
    mesh=plsc.VectorSubcoreMesh(core_axis_name="c", subcore_axis_name="s"),
    scratch_types=[
        pltpu.VMEM((BPW,), jnp.int32),
        pltpu.VMEM((BPW, NP), _F32),
        pltpu.VMEM((BPW, NP), _F32),
        pltpu.SemaphoreType.DMA,
    ],
)(_gather_body)


M_BLOCK = 2048


def _mlp_body(x1_ref, hour_ref, wh_ref, b1_ref, w2_ref, b2_ref, w3_ref, b3_ref,
              out_ref):
    a = x1_ref[...] + hour_ref[...] * wh_ref[...] + b1_ref[...]
    a = jnp.maximum(a, 0.0)
    h = lax.dot_general(a, w2_ref[...], (((1,), (0,)), ((), ())),
                        preferred_element_type=_F32, precision=_HIGH)
    h = jnp.maximum(h + b2_ref[...], 0.0)
    y = jnp.sum(h * w3_ref[...], axis=1, keepdims=True) + b3_ref[...]
    out_ref[...] = jax.nn.sigmoid(y)


def _run_mlp(x1, hour_col, wh, b1p, w2p, b2p, w3p, b3p):
    full = lambda a: pl.BlockSpec(a.shape, lambda i: (0, 0))
    return pl.pallas_call(
        _mlp_body,
        grid=(B // M_BLOCK,),
        in_specs=[
            pl.BlockSpec((M_BLOCK, NP), lambda i: (i, 0)),
            pl.BlockSpec((M_BLOCK, 1), lambda i: (i, 0)),
            full(wh), full(b1p), full(w2p), full(b2p), full(w3p), full(b3p),
        ],
        out_specs=pl.BlockSpec((M_BLOCK, 1), lambda i: (i, 0)),
        out_shape=jax.ShapeDtypeStruct((B, 1), _F32),
    )(x1, hour_col, wh, b1p, w2p, b2p, w3p, b3p)


def kernel(idx_0, idx_1, idx_2, idx_3, idx_4, idx_5, idx_6, idx_7, idx_8,
           idx_9, idx_10, hour,
           table_0, table_1, table_2, table_3, table_4, table_5, table_6,
           table_7, table_8, table_9, table_10,
           W1, b1, W2, b2, W3, b3):
    idxs = (idx_0, idx_1, idx_2, idx_3, idx_4, idx_5, idx_6, idx_7, idx_8,
            idx_9, idx_10)
    tables = (table_0, table_1, table_2, table_3, table_4, table_5, table_6,
              table_7, table_8, table_9, table_10)

    w1t = jnp.pad(W1, ((0, NP - W1.shape[0]), (0, 0))).T
    w_slices = tuple(w1t[OFFS[t]:OFFS[t] + DIMS[t], :] for t in range(NT))
    wh = w1t[OFFS[NT - 1] + 1][None, :]
    b1p = jnp.pad(b1, (0, NP - b1.shape[0]))[None, :]
    w2p = jnp.pad(W2.T, ((0, NP - W2.shape[1]), (0, NP - W2.shape[0])))
    b2p = jnp.pad(b2, (0, NP - b2.shape[0]))[None, :]
    w3p = jnp.pad(W3, ((0, 0), (0, NP - W3.shape[1])))
    b3p = b3[None, :]

    proj = _run_projection(tables, w_slices)
    x1 = _gather_sum(*proj, *idxs)
    return _run_mlp(x1, hour[:, None], wh, b1p, w2p, b2p, w3p, b3p)


# speedup vs baseline: 1.4414x; 1.4414x over previous
import functools

import jax
import jax.numpy as jnp
from jax import lax
from jax.experimental import pallas as pl
from jax.experimental.pallas import tpu as pltpu
from jax.experimental.pallas import tpu_sc as plsc

B = 16384
VOCABS = (1000, 8, 30, 40, 8, 4, 20, 20, 4, 10000, 7)
DIMS = (100, 1, 3, 4, 1, 1, 2, 2, 1, 1000, 1)
OFFS = (0, 100, 101, 104, 108, 109, 110, 112, 114, 115, 1115)
NT = 11
NP = 16

NC, NS = 2, 16
NW = NC * NS
BPW = B // NW

T9_BLOCK = 2000
T9_GRID = VOCABS[9] // T9_BLOCK

_F32 = jnp.float32
_HIGH = lax.Precision.HIGHEST


def _proj_body(*refs):
    t_refs = refs[:NT]
    w_refs = refs[NT:2 * NT]
    p_refs = refs[2 * NT:]
    step = pl.program_id(0)

    def project(t):
        tbl = t_refs[t][...]
        d = DIMS[t]
        if d >= 8:
            return lax.dot_general(tbl, w_refs[t][...], (((1,), (0,)), ((), ())),
                                   preferred_element_type=_F32, precision=_HIGH)
        acc = tbl[:, 0:1] * w_refs[t][0:1, :]
        for k in range(1, d):
            acc = acc + tbl[:, k:k + 1] * w_refs[t][k:k + 1, :]
        return acc

    p_refs[9][...] = project(9)

    @pl.when(step == 0)
    def _():
        for t in range(NT):
            if t != 9:
                p_refs[t][...] = project(t)


def _run_projection(tables, w_slices):
    in_specs = []
    for t in range(NT):
        if t == 9:
            in_specs.append(pl.BlockSpec((T9_BLOCK, DIMS[9]), lambda i: (i, 0)))
        else:
            in_specs.append(pl.BlockSpec(tables[t].shape, lambda i: (0, 0)))
    for t in range(NT):
        in_specs.append(pl.BlockSpec(w_slices[t].shape, lambda i: (0, 0)))
    out_specs = []
    out_shapes = []
    for t in range(NT):
        out_shapes.append(jax.ShapeDtypeStruct((VOCABS[t], NP), _F32))
        if t == 9:
            out_specs.append(pl.BlockSpec((T9_BLOCK, NP), lambda i: (i, 0)))
        else:
            out_specs.append(pl.BlockSpec((VOCABS[t], NP), lambda i: (0, 0)))
    return pl.pallas_call(
        _proj_body,
        grid=(T9_GRID,),
        in_specs=in_specs,
        out_specs=out_specs,
        out_shape=out_shapes,
    )(*tables, *w_slices)


_UNROLL = 8


def _gather_body(*refs):
    p_refs = refs[:NT]
    i_refs = refs[NT:2 * NT]
    out_hbm = refs[2 * NT]
    idx_v, acc_v, rows_v, sem = refs[2 * NT + 1:]

    wid = lax.axis_index("s") * NC + lax.axis_index("c")
    base = wid * BPW

    pltpu.sync_copy(i_refs[0].at[pl.ds(base, BPW)], idx_v)
    pltpu.async_copy(p_refs[0].at[idx_v], acc_v, sem).wait()

    for t in range(1, NT):
        pltpu.sync_copy(i_refs[t].at[pl.ds(base, BPW)], idx_v)
        pltpu.async_copy(p_refs[t].at[idx_v], rows_v, sem).wait()

        def body(i, _):
            b0 = i * _UNROLL
            for u in range(_UNROLL):
                plsc.addupdate(acc_v.at[b0 + u], rows_v[b0 + u])
            return 0

        lax.fori_loop(0, BPW // _UNROLL, body, 0)

    pltpu.sync_copy(acc_v, out_hbm.at[pl.ds(base, BPW)])


@functools.cache
def _gather_sum():
    return pl.kernel(
        _gather_body,
        out_type=jax.ShapeDtypeStruct((B, NP), _F32),
        mesh=plsc.VectorSubcoreMesh(core_axis_name="c", subcore_axis_name="s"),
        scratch_types=[
            pltpu.VMEM((BPW,), jnp.int32),
            pltpu.VMEM((BPW, NP), _F32),
            pltpu.VMEM((BPW, NP), _F32),
            pltpu.SemaphoreType.DMA,
        ],
        compiler_params=pltpu.CompilerParams(use_tc_tiling_on_sc=False),
    )


M_BLOCK = 2048


def _mlp_body(x1_ref, hour_ref, wh_ref, b1_ref, w2_ref, b2_ref, w3_ref, b3_ref,
              out_ref):
    a = x1_ref[...] + hour_ref[...] * wh_ref[...] + b1_ref[...]
    a = jnp.maximum(a, 0.0)
    h = lax.dot_general(a, w2_ref[...], (((1,), (0,)), ((), ())),
                        preferred_element_type=_F32, precision=_HIGH)
    h = jnp.maximum(h + b2_ref[...], 0.0)
    y = jnp.sum(h * w3_ref[...], axis=1, keepdims=True) + b3_ref[...]
    out_ref[...] = jax.nn.sigmoid(y)


def _run_mlp(x1, hour_col, wh, b1p, w2p, b2p, w3p, b3p):
    full = lambda a: pl.BlockSpec(a.shape, lambda i: (0, 0))
    return pl.pallas_call(
        _mlp_body,
        grid=(B // M_BLOCK,),
        in_specs=[
            pl.BlockSpec((M_BLOCK, NP), lambda i: (i, 0)),
            pl.BlockSpec((M_BLOCK, 1), lambda i: (i, 0)),
            full(wh), full(b1p), full(w2p), full(b2p), full(w3p), full(b3p),
        ],
        out_specs=pl.BlockSpec((M_BLOCK, 1), lambda i: (i, 0)),
        out_shape=jax.ShapeDtypeStruct((B, 1), _F32),
    )(x1, hour_col, wh, b1p, w2p, b2p, w3p, b3p)


def kernel(idx_0, idx_1, idx_2, idx_3, idx_4, idx_5, idx_6, idx_7, idx_8,
           idx_9, idx_10, hour,
           table_0, table_1, table_2, table_3, table_4, table_5, table_6,
           table_7, table_8, table_9, table_10,
           W1, b1, W2, b2, W3, b3):
    idxs = (idx_0, idx_1, idx_2, idx_3, idx_4, idx_5, idx_6, idx_7, idx_8,
            idx_9, idx_10)
    tables = (table_0, table_1, table_2, table_3, table_4, table_5, table_6,
              table_7, table_8, table_9, table_10)

    w1t = jnp.pad(W1, ((0, NP - W1.shape[0]), (0, 0))).T
    w_slices = tuple(w1t[OFFS[t]:OFFS[t] + DIMS[t], :] for t in range(NT))
    wh = w1t[OFFS[NT - 1] + 1][None, :]
    b1p = jnp.pad(b1, (0, NP - b1.shape[0]))[None, :]
    w2p = jnp.pad(W2.T, ((0, NP - W2.shape[1]), (0, NP - W2.shape[0])))
    b2p = jnp.pad(b2, (0, NP - b2.shape[0]))[None, :]
    w3p = jnp.pad(W3, ((0, 0), (0, NP - W3.shape[1])))
    b3p = b3[None, :]

    proj = _run_projection(tables, w_slices)
    x1 = _gather_sum()(*proj, *idxs)
    return _run_mlp(x1, hour[:, None], wh, b1p, w2p, b2p, w3p, b3p)


# speedup vs baseline: 7.9051x; 5.4843x over previous
import functools

import jax
import jax.numpy as jnp
from jax import lax
from jax.experimental import pallas as pl
from jax.experimental.pallas import tpu as pltpu
from jax.experimental.pallas import tpu_sc as plsc

B = 16384
VOCABS = (1000, 8, 30, 40, 8, 4, 20, 20, 4, 10000, 7)
DIMS = (100, 1, 3, 4, 1, 1, 2, 2, 1, 1000, 1)
OFFS = (0, 100, 101, 104, 108, 109, 110, 112, 114, 115, 1115)
NT = 11
NP = 16

NC, NS = 2, 16
NW = NC * NS
NSM = 10
BPW = B // NW

T9_KBLK = 200
T9_GRID = DIMS[9] // T9_KBLK

P9W = 128

_F32 = jnp.float32
_HIGH = lax.Precision.HIGHEST

_IDXROW = {t: k for k, t in enumerate((0, 1, 2, 3, 4, 5, 6, 7, 8, 10))}
_LOOKUPS = (
    (0, None, 0),
    (2, 3, 40),
    (6, 7, 24),
    (1, 4, 8),
    (5, 8, 8),
    (10, None, 0),
)


def _round8(n):
    return (n + 7) // 8 * 8


_LOFF = []
_o = 0
for _a, _b, _vbp in _LOOKUPS:
    _LOFF.append(_o)
    _o += _round8(VOCABS[_a] * (_vbp if _b is not None else 1)
                  if _b is not None else VOCABS[_a])
WH_ROW = _o
PS_ROWS = _o + 8


def _proj_body(*refs):
    t_refs = refs[:NT]
    w1t_ref = refs[NT]
    b1_ref = refs[NT + 1]
    p9_ref, ps_ref = refs[NT + 2:]
    step = pl.program_id(0)

    def wslice(lo, d):
        w = w1t_ref[lo:lo + d, :]
        return jnp.concatenate([w, jnp.zeros((d, NP - 10), _F32)], axis=1)

    def project(t):
        return lax.dot_general(t_refs[t][...], wslice(OFFS[t], DIMS[t]),
                               (((0,), (0,)), ((), ())),
                               preferred_element_type=_F32)

    w9 = w1t_ref[pl.ds(OFFS[9] + step * T9_KBLK, T9_KBLK), :]
    w9 = jnp.concatenate([w9, jnp.zeros((T9_KBLK, NP - 10), _F32)], axis=1)
    partial = lax.dot_general(t_refs[9][...], w9, (((0,), (0,)), ((), ())),
                              preferred_element_type=_F32)

    @pl.when(step == 0)
    def _():
        p9_ref[:, 0:16] = partial

    @pl.when(step != 0)
    def _():
        p9_ref[:, 0:16] += partial

    @pl.when(step == 0)
    def _():
        for li, (a, b, vbp) in enumerate(_LOOKUPS):
            pa = project(a)
            if a == 10:
                pa = pa + b1_ref[...]
            off = _LOFF[li]
            if b is None:
                ps_ref[off:off + VOCABS[a], :] = pa
            else:
                pb = project(b)
                for j in range(VOCABS[a]):
                    ps_ref[off + j * vbp:off + j * vbp + VOCABS[b], :] = (
                        pa[j:j + 1, :] + pb)
        ps_ref[WH_ROW:WH_ROW + 1, :] = wslice(1116, 1)


def _run_projection(tables_t, w1t, b1p):
    in_specs = []
    for t in range(NT):
        if t == 9:
            in_specs.append(
                pl.BlockSpec((T9_KBLK, VOCABS[9]), lambda i: (i, 0)))
        else:
            in_specs.append(pl.BlockSpec(tables_t[t].shape, lambda i: (0, 0)))
    in_specs.append(pl.BlockSpec(w1t.shape, lambda i: (0, 0)))
    in_specs.append(pl.BlockSpec((1, NP), lambda i: (0, 0)))
    return pl.pallas_call(
        _proj_body,
        grid=(T9_GRID,),
        in_specs=in_specs,
        out_specs=[
            pl.BlockSpec((VOCABS[9], P9W), lambda i: (0, 0)),
            pl.BlockSpec((PS_ROWS, NP), lambda i: (0, 0)),
        ],
        out_shape=[
            jax.ShapeDtypeStruct((VOCABS[9], P9W), _F32),
            jax.ShapeDtypeStruct((PS_ROWS, NP), _F32),
        ],
    )(*tables_t, w1t, b1p)


_NCHUNK = 4
_CHUNK = BPW // _NCHUNK
_NBUF = 2


def _gather_body(*refs):
    p9_hbm, ps_hbm = refs[0], refs[1]
    small_idx_hbm = refs[2:2 + NSM]
    hb_hbm = refs[2 + NSM]
    idx9_hbm = refs[3 + NSM]
    out_hbm = refs[4 + NSM]
    idx_v, idx9_v, ps_v, rows9_v, acc_v, sem, sem2, semi, semg = \
        refs[5 + NSM:]

    wid = lax.axis_index("s") * NC + lax.axis_index("c")
    base = wid * BPW

    cp9 = pltpu.async_copy(idx9_hbm.at[pl.ds(base, BPW)], idx9_v, semi)
    cps = []
    for k in range(NSM):
        cps.append(pltpu.async_copy(small_idx_hbm[k].at[pl.ds(base, BPW)],
                                    idx_v.at[k], sem))
    cps.append(pltpu.async_copy(hb_hbm.at[pl.ds(base, BPW)],
                                idx_v.at[NSM], sem))
    cp_ps = pltpu.async_copy(ps_hbm, ps_v, sem2)
    cp9.wait()

    def fire(c):
        return pltpu.async_copy(
            p9_hbm.at[idx9_v.at[pl.ds(c * _CHUNK, _CHUNK)]],
            rows9_v.at[c % _NBUF], semg[c % _NBUF])

    gs = {c: fire(c) for c in range(_NBUF)}
    for cp in cps:
        cp.wait()
    cp_ps.wait()
    whv = ps_v[WH_ROW]

    for c in range(_NCHUNK):
        gs[c].wait()

        def body(g, _, c=c):
            b0 = c * _CHUNK + g * 16
            r0 = g * 16
            hv = plsc.bitcast(idx_v[NSM, pl.ds(b0, 16)], _F32)
            iv = [idx_v[k, pl.ds(b0, 16)] for k in range(NSM)]
            lidx = []
            for li, (a, b, vbp) in enumerate(_LOOKUPS):
                v = iv[_IDXROW[a]]
                if b is not None:
                    v = v * vbp + iv[_IDXROW[b]]
                lidx.append(v)
            for u in range(16):
                b_ = b0 + u
                row = rows9_v[c % _NBUF, r0 + u, 0:16] + hv[u] * whv
                for li in range(len(_LOOKUPS)):
                    row = row + ps_v[lidx[li][u] + _LOFF[li]]
                acc_v[pl.ds(b_ * NP, NP)] = row
            return 0

        lax.fori_loop(0, _CHUNK // 16, body, 0)
        if c + _NBUF < _NCHUNK:
            gs[c + _NBUF] = fire(c + _NBUF)

    pltpu.sync_copy(acc_v, out_hbm.at[pl.ds(base * NP, BPW * NP)])


@functools.cache
def _gather_sum():
    return pl.kernel(
        _gather_body,
        out_type=jax.ShapeDtypeStruct((B * NP,), _F32),
        mesh=plsc.VectorSubcoreMesh(core_axis_name="c", subcore_axis_name="s"),
        scratch_types=[
            pltpu.VMEM((NSM + 1, BPW), jnp.int32),
            pltpu.VMEM((BPW,), jnp.int32),
            pltpu.VMEM((PS_ROWS, NP), _F32),
            pltpu.VMEM((_NBUF, _CHUNK, P9W), _F32),
            pltpu.VMEM((BPW * NP,), _F32),
            pltpu.SemaphoreType.DMA,
            pltpu.SemaphoreType.DMA,
            pltpu.SemaphoreType.DMA,
            [pltpu.SemaphoreType.DMA] * _NBUF,
        ],
        compiler_params=pltpu.CompilerParams(use_tc_tiling_on_sc=False,
                                             needs_layout_passes=False),
    )


M_BLOCK = 512


def _mlp_body(x_ref, w2_ref, b2_ref, w3_ref, b3_ref, out_ref):
    a = jnp.maximum(x_ref[...], 0.0)
    h = lax.dot_general(a, w2_ref[...], (((1,), (0,)), ((), ())),
                        preferred_element_type=_F32, precision=_HIGH)
    h = jnp.maximum(h + b2_ref[...], 0.0)
    y = lax.dot_general(h, w3_ref[...], (((1,), (0,)), ((), ())),
                        preferred_element_type=_F32, precision=_HIGH)
    out_ref[...] = jax.nn.sigmoid(y + b3_ref[...])


def _run_mlp(xp, w2bd, b2t, w3bd, b3t):
    full = lambda a: pl.BlockSpec(a.shape, lambda i: (0, 0))
    return pl.pallas_call(
        _mlp_body,
        grid=(B // 8 // M_BLOCK,),
        in_specs=[
            pl.BlockSpec((M_BLOCK, P9W), lambda i: (i, 0)),
            full(w2bd), full(b2t), full(w3bd), full(b3t),
        ],
        out_specs=pl.BlockSpec((M_BLOCK, 8), lambda i: (i, 0)),
        out_shape=jax.ShapeDtypeStruct((B // 8, 8), _F32),
    )(xp, w2bd, b2t, w3bd, b3t)


def kernel(idx_0, idx_1, idx_2, idx_3, idx_4, idx_5, idx_6, idx_7, idx_8,
           idx_9, idx_10, hour,
           table_0, table_1, table_2, table_3, table_4, table_5, table_6,
           table_7, table_8, table_9, table_10,
           W1, b1, W2, b2, W3, b3):
    idxs = (idx_0, idx_1, idx_2, idx_3, idx_4, idx_5, idx_6, idx_7, idx_8,
            idx_9, idx_10)
    tables = (table_0, table_1, table_2, table_3, table_4, table_5, table_6,
              table_7, table_8, table_9, table_10)

    w1t = W1.T
    b1p = jnp.pad(b1, (0, NP - b1.shape[0]))[None, :]
    eye8 = jnp.eye(8, dtype=_F32)
    w2bd = jnp.kron(eye8, jnp.pad(W2.T, ((0, 6), (0, 11))))
    b2t = jnp.tile(jnp.pad(b2, (0, 11)), 8)[None, :]
    w3bd = jnp.kron(eye8, jnp.pad(W3, ((0, 0), (0, 11))).T)
    b3t = jnp.tile(b3, 8)[None, :]
    hour_bits = lax.bitcast_convert_type(hour, jnp.int32)
    tables_t = tuple(t.T for t in tables)

    p9w, ps = _run_projection(tables_t, w1t, b1p)
    x1v = _gather_sum()(p9w, ps,
                        *(idxs[t] for t in (0, 1, 2, 3, 4, 5, 6, 7, 8, 10)),
                        hour_bits, idx_9)
    y8 = _run_mlp(x1v.reshape(B // 8, P9W), w2bd, b2t, w3bd, b3t)
    return y8.reshape(B, 1)
